# baseline (speedup 1.0000x reference)
import functools

import jax
import jax.numpy as jnp
from jax import lax
from jax.experimental import pallas as pl
from jax.experimental.pallas import tpu as pltpu
from jax.experimental.pallas import tpu_sc as plsc

T, D_MODEL, D_FF, E, K_BEST = 2048, 1024, 2048, 8, 2
TILE = 64
NT = (T * K_BEST) // TILE + E
NR = NT * TILE
NW = 32
TPW = T // NW
CHUNK = 512
NSLOT = 2


GW = 128


def _router_body(x_ref, wg_ref, d0_ref, d1_ref, g0_ref, g1_ref, meta_ref):
    xv = x_ref[...]
    logits = jnp.dot(xv, wg_ref[...], preferred_element_type=jnp.float32)

    lane = lax.broadcasted_iota(jnp.int32, (T, E), 1)
    m1 = jnp.max(logits, axis=1, keepdims=True)
    i1 = jnp.min(jnp.where(logits == m1, lane, E), axis=1, keepdims=True)
    masked = jnp.where(lane == i1, -jnp.inf, logits)
    m2 = jnp.max(masked, axis=1, keepdims=True)
    i2 = jnp.min(jnp.where(masked == m2, lane, E), axis=1, keepdims=True)

    g1 = 1.0 / (1.0 + jnp.exp(m2 - m1))
    g2 = 1.0 - g1

    c1 = (lane == i1).astype(jnp.float32)
    c2 = (lane == i2).astype(jnp.float32)

    r = lax.broadcasted_iota(jnp.int32, (CHUNK, CHUNK), 0)
    c = lax.broadcasted_iota(jnp.int32, (CHUNK, CHUNK), 1)
    tril = (c < r).astype(jnp.float32)

    def excl_cumsum(cm):
        parts = []
        carry = jnp.zeros((1, E), jnp.float32)
        for ci in range(T // CHUNK):
            blk = cm[ci * CHUNK:(ci + 1) * CHUNK, :]
            parts.append(jnp.dot(tril, blk, preferred_element_type=jnp.float32)
                         + carry)
            carry = carry + jnp.sum(blk, axis=0, keepdims=True)
        return jnp.concatenate(parts, axis=0), carry

    e1, cnt1 = excl_cumsum(c1)
    e2, cnt2 = excl_cumsum(c2)
    e2 = e2 + cnt1
    counts = cnt1 + cnt2

    pci = (counts.astype(jnp.int32) + (TILE - 1)) & ~jnp.int32(TILE - 1)
    pcf = pci.astype(jnp.float32)
    er = lax.broadcasted_iota(jnp.int32, (E, E), 0)
    ec = lax.broadcasted_iota(jnp.int32, (E, E), 1)
    triu = (er < ec).astype(jnp.float32)
    off = jnp.dot(pcf, triu, preferred_element_type=jnp.float32)

    d0_ref[...] = jnp.sum(c1 * (e1 + off), axis=1, keepdims=True).astype(jnp.int32)
    d1_ref[...] = jnp.sum(c2 * (e2 + off), axis=1, keepdims=True).astype(jnp.int32)
    g0_ref[...] = jnp.broadcast_to(g1, (T, GW))
    g1_ref[...] = jnp.broadcast_to(g2, (T, GW))

    bnd = off + pcf
    jt = (lax.broadcasted_iota(jnp.int32, (1, NT), 1) * TILE).astype(jnp.float32)
    total = bnd[0:1, E - 1:E]
    texp = jnp.zeros((1, NT), jnp.int32)
    elast = jnp.zeros((1, 1), jnp.int32)
    isfirst = jnp.zeros((1, NT), jnp.int32)
    segplus = jnp.zeros((1, NT), jnp.int32)
    for e in range(E):
        b = bnd[0:1, e:e + 1]
        o = off[0:1, e:e + 1]
        pe = pcf[0:1, e:e + 1]
        texp = texp + (b <= jt).astype(jnp.int32)
        elast = elast + (b <= total - 1.0).astype(jnp.int32)
        isfirst = isfirst | ((jt == o) & (pe > 0)).astype(jnp.int32)
        segplus = segplus + ((o <= jt) & (pe > 0)).astype(jnp.int32)
    act = (jt < total).astype(jnp.int32)
    texp = jnp.where(act == 1, jnp.minimum(texp, E - 1), elast)
    isfirst = isfirst * act

    slot = jnp.zeros((1, NT), jnp.int32)
    islot = jnp.zeros((1, NT), jnp.int32)
    for s in range(E):
        slot = slot + jnp.where(segplus == s + 1, s % NSLOT, 0)
        islot = islot + jnp.where(segplus == s + 1, (s + NSLOT - 1) % NSLOT, 0)

    pfe1 = jnp.zeros((1, NT), jnp.int32)
    pfe2 = jnp.zeros((1, NT), jnp.int32)
    nxt_list = []
    for e in range(E):
        nxt = jnp.full((1, 1), E, jnp.int32)
        for e2 in range(E - 1, e, -1):
            nxt = jnp.where(pcf[0:1, e2:e2 + 1] > 0, e2, nxt)
        nxt_list.append(nxt)
    for e in range(E):
        nxt2 = jnp.full((1, 1), E, jnp.int32)
        for e2 in range(E):
            nxt2 = jnp.where(nxt_list[e] == e2, nxt_list[e2], nxt2)
        pfe1 = pfe1 + jnp.where(texp == e, nxt_list[e], 0)
        pfe2 = pfe2 + jnp.where(texp == e, nxt2, 0)
    pfv1 = isfirst * (pfe1 < E).astype(jnp.int32)
    pfv2 = isfirst * (pfe2 < E).astype(jnp.int32)
    pfe1 = jnp.minimum(pfe1, E - 1)
    pfe2 = jnp.minimum(pfe2, E - 1)

    meta_ref[0:1, :] = texp
    meta_ref[1:2, :] = act
    meta_ref[2:3, :] = isfirst
    meta_ref[3:4, :] = slot
    meta_ref[4:5, :] = pfe2 if NSLOT >= 3 else pfe1
    meta_ref[5:6, :] = pfv2 if NSLOT >= 3 else pfv1
    meta_ref[6:7, :] = islot
    meta_ref[7:8, :] = pfe1
    meta_ref[8:9, :] = pfv1


def _router(x, wg):
    return pl.pallas_call(
        _router_body,
        out_shape=(
            jax.ShapeDtypeStruct((T, 1), jnp.int32),
            jax.ShapeDtypeStruct((T, 1), jnp.int32),
            jax.ShapeDtypeStruct((T, GW), jnp.float32),
            jax.ShapeDtypeStruct((T, GW), jnp.float32),
            jax.ShapeDtypeStruct((9, NT), jnp.int32),
        ),
    )(x, wg)


def _dispatch_body(x_hbm, g0_hbm, g1_hbm, d0_hbm, d1_hbm, xg_hbm, gp_hbm,
                   idx0, idx1, rows, gr0, gr1, ssem, scs):
    wid = lax.axis_index("s") * 2 + lax.axis_index("c")
    base = wid * TPW
    stage = (
        pltpu.async_copy(d0_hbm.at[pl.ds(base, TPW)], idx0, ssem),
        pltpu.async_copy(d1_hbm.at[pl.ds(base, TPW)], idx1, ssem),
        pltpu.async_copy(x_hbm.at[pl.ds(base, TPW)], rows, ssem),
        pltpu.async_copy(g0_hbm.at[pl.ds(base, TPW)], gr0, ssem),
        pltpu.async_copy(g1_hbm.at[pl.ds(base, TPW)], gr1, ssem),
    )
    for c in stage:
        c.wait()
    scat = (
        pltpu.async_copy(rows, xg_hbm.at[idx0], scs),
        pltpu.async_copy(rows, xg_hbm.at[idx1], scs),
        pltpu.async_copy(gr0, gp_hbm.at[idx0], scs),
        pltpu.async_copy(gr1, gp_hbm.at[idx1], scs),
    )
    for c in scat:
        c.wait()


def _dispatch(x, g0, g1, d0, d1):
    fn = functools.partial(
        pl.kernel,
        out_type=(
            jax.ShapeDtypeStruct((NR, D_MODEL), jnp.float32),
            jax.ShapeDtypeStruct((NR, GW), jnp.float32),
        ),
        mesh=plsc.VectorSubcoreMesh(core_axis_name="c", subcore_axis_name="s"),
        scratch_types=[
            pltpu.VMEM((TPW,), jnp.int32),
            pltpu.VMEM((TPW,), jnp.int32),
            pltpu.VMEM((TPW, D_MODEL), jnp.float32),
            pltpu.VMEM((TPW, GW), jnp.float32),
            pltpu.VMEM((TPW, GW), jnp.float32),
            pltpu.SemaphoreType.DMA,
            pltpu.SemaphoreType.DMA,
        ],
    )(_dispatch_body)
    return fn(x, g0, g1, d0, d1)


def _ffn_body(te_ref, ta_ref, if_ref, sl_ref, pfe2_ref, pfv2_ref,
              il_ref, pfe1_ref, pfv1_ref,
              xg_ref, w1_hbm, w2_hbm, gp_ref, out_ref,
              w1b, w2b, s1, s2):
    i = pl.program_id(0)

    def start_fetch(expert, slot):
        pltpu.make_async_copy(w1_hbm.at[expert], w1b.at[slot],
                              s1.at[slot]).start()
        pltpu.make_async_copy(w2_hbm.at[expert], w2b.at[slot],
                              s2.at[slot]).start()

    def wait_fetch(slot):
        pltpu.make_async_copy(w1_hbm.at[0], w1b.at[slot], s1.at[slot]).wait()
        pltpu.make_async_copy(w2_hbm.at[0], w2b.at[slot], s2.at[slot]).wait()

    @pl.when(i == 0)
    def _():
        start_fetch(te_ref[0], 0)

    if NSLOT >= 3:
        @pl.when((i == 0) & (pfv1_ref[0] == 1))
        def _():
            start_fetch(pfe1_ref[0], 1)

    for sl in range(NSLOT):
        @pl.when((pfv2_ref[i] == 1) & (il_ref[i] == sl))
        def _(sl=sl):
            start_fetch(pfe2_ref[i], sl)

    for sl in range(NSLOT):
        @pl.when((if_ref[i] == 1) & (sl_ref[i] == sl))
        def _(sl=sl):
            wait_fetch(sl)

    def compute(slot):
        h = jnp.maximum(
            jnp.dot(xg_ref[...], w1b[slot],
                    preferred_element_type=jnp.float32), 0.0)
        out = jnp.dot(h, w2b[slot], preferred_element_type=jnp.float32)
        out_ref[...] = out * gp_ref[:, 0:1]

    for sl in range(NSLOT):
        @pl.when((ta_ref[i] == 1) & (sl_ref[i] == sl))
        def _(sl=sl):
            compute(sl)


def _ffn(meta, xg, w1, w2, gp):
    grid_spec = pltpu.PrefetchScalarGridSpec(
        num_scalar_prefetch=9,
        grid=(NT,),
        in_specs=[
            pl.BlockSpec((TILE, D_MODEL), lambda i, *_: (i, 0)),
            pl.BlockSpec(memory_space=pltpu.MemorySpace.HBM),
            pl.BlockSpec(memory_space=pltpu.MemorySpace.HBM),
            pl.BlockSpec((TILE, GW), lambda i, *_: (i, 0)),
        ],
        out_specs=pl.BlockSpec((TILE, D_MODEL), lambda i, *_: (i, 0)),
        scratch_shapes=[
            pltpu.VMEM((NSLOT, D_MODEL, D_FF), jnp.float32),
            pltpu.VMEM((NSLOT, D_FF, D_MODEL), jnp.float32),
            pltpu.SemaphoreType.DMA((NSLOT,)),
            pltpu.SemaphoreType.DMA((NSLOT,)),
        ],
    )
    return pl.pallas_call(
        _ffn_body,
        grid_spec=grid_spec,
        out_shape=jax.ShapeDtypeStruct((NR, D_MODEL), jnp.float32),
    )(*[meta[r] for r in range(9)], xg, w1, w2, gp)


SUB = 64


SUBC = 16


def _combine_body(out_hbm, d0_hbm, d1_hbm, y_hbm,
                  i0a, i0b, i1a, i1b, r0a, r0b, r1a, r1b,
                  isa, isb, gsa, gsb, osa, osb):
    wid = lax.axis_index("s") * 2 + lax.axis_index("c")
    idx0 = (i0a, i0b)
    idx1 = (i1a, i1b)
    r0 = (r0a, r0b)
    r1 = (r1a, r1b)
    isem = (isa, isb)
    gsem = (gsa, gsb)
    osem = (osa, osb)
    nch = TPW // SUBC

    def stage_idx(k):
        base = wid * TPW + k * SUBC
        p = k % 2
        return (pltpu.async_copy(d0_hbm.at[pl.ds(base, SUBC)], idx0[p], isem[p]),
                pltpu.async_copy(d1_hbm.at[pl.ds(base, SUBC)], idx1[p], isem[p]))

    def start_gather(k, staged):
        p = k % 2
        for c in staged:
            c.wait()
        return (pltpu.async_copy(out_hbm.at[idx0[p]], r0[p], gsem[p]),
                pltpu.async_copy(out_hbm.at[idx1[p]], r1[p], gsem[p]))

    st = stage_idx(0)
    g = start_gather(0, st)
    st_next = stage_idx(1)
    gathers = {0: g}
    for k in range(nch):
        p = k % 2
        if k + 1 < nch:
            if k >= 1:
                pltpu.make_async_copy(r0[1 - p], y_hbm.at[pl.ds(0, SUBC)],
                                      osem[1 - p]).wait()
            gathers[k + 1] = start_gather(k + 1, st_next)
        for c in gathers[k]:
            c.wait()
        if k + 2 < nch:
            st_next = stage_idx(k + 2)

        def col(v, _):
            s = pl.ds(v * 16, 16)
            for j in range(SUBC):
                r0[p][j, s] = r0[p][j, s] + r1[p][j, s]
            return 0

        lax.fori_loop(0, D_MODEL // 16, col, 0, unroll=2)
        base = wid * TPW + k * SUBC
        pltpu.async_copy(r0[p], y_hbm.at[pl.ds(base, SUBC)], osem[p])
    for p in ((nch - 2) % 2, (nch - 1) % 2):
        pltpu.make_async_copy(r0[p], y_hbm.at[pl.ds(0, SUBC)], osem[p]).wait()


def _combine(out, d0, d1):
    fn = functools.partial(
        pl.kernel,
        out_type=jax.ShapeDtypeStruct((T, D_MODEL), jnp.float32),
        mesh=plsc.VectorSubcoreMesh(core_axis_name="c", subcore_axis_name="s"),
        scratch_types=[
            pltpu.VMEM((SUBC,), jnp.int32),
            pltpu.VMEM((SUBC,), jnp.int32),
            pltpu.VMEM((SUBC,), jnp.int32),
            pltpu.VMEM((SUBC,), jnp.int32),
            pltpu.VMEM((SUBC, D_MODEL), jnp.float32),
            pltpu.VMEM((SUBC, D_MODEL), jnp.float32),
            pltpu.VMEM((SUBC, D_MODEL), jnp.float32),
            pltpu.VMEM((SUBC, D_MODEL), jnp.float32),
            pltpu.SemaphoreType.DMA,
            pltpu.SemaphoreType.DMA,
            pltpu.SemaphoreType.DMA,
            pltpu.SemaphoreType.DMA,
            pltpu.SemaphoreType.DMA,
            pltpu.SemaphoreType.DMA,
        ],
    )(_combine_body)
    return fn(out, d0, d1)


def kernel(x, Wg, W1, W2):
    d0, d1, g0, g1, meta = _router(x, Wg)
    d0 = d0.reshape(T)
    d1 = d1.reshape(T)
    xg, gp = _dispatch(x, g0, g1, d0, d1)
    out = _ffn(meta, xg, W1, W2, gp)
    return _combine(out, d0, d1)


# speedup vs baseline: 1.3643x; 1.3643x over previous
import functools

import jax
import jax.numpy as jnp
from jax import lax
from jax.experimental import pallas as pl
from jax.experimental.pallas import tpu as pltpu
from jax.experimental.pallas import tpu_sc as plsc

T, D_MODEL, D_FF, E, K_BEST = 2048, 1024, 2048, 8, 2
TILE = 128
NT = (T * K_BEST) // TILE + E
NR = NT * TILE
NW = 32
TPW = T // NW
CHUNK = 512
NSLOT = 2


GW = 128


def _router_body(x_ref, wg_ref, d0_ref, d1_ref, g0_ref, g1_ref, meta_ref):
    xv = x_ref[...]
    logits = jnp.dot(xv, wg_ref[...], preferred_element_type=jnp.float32)

    lane = lax.broadcasted_iota(jnp.int32, (T, E), 1)
    m1 = jnp.max(logits, axis=1, keepdims=True)
    i1 = jnp.min(jnp.where(logits == m1, lane, E), axis=1, keepdims=True)
    masked = jnp.where(lane == i1, -jnp.inf, logits)
    m2 = jnp.max(masked, axis=1, keepdims=True)
    i2 = jnp.min(jnp.where(masked == m2, lane, E), axis=1, keepdims=True)

    g1 = 1.0 / (1.0 + jnp.exp(m2 - m1))
    g2 = 1.0 - g1

    c1 = (lane == i1).astype(jnp.float32)
    c2 = (lane == i2).astype(jnp.float32)

    r = lax.broadcasted_iota(jnp.int32, (CHUNK, CHUNK), 0)
    c = lax.broadcasted_iota(jnp.int32, (CHUNK, CHUNK), 1)
    tril = (c < r).astype(jnp.float32)

    def excl_cumsum(cm):
        parts = []
        carry = jnp.zeros((1, E), jnp.float32)
        for ci in range(T // CHUNK):
            blk = cm[ci * CHUNK:(ci + 1) * CHUNK, :]
            parts.append(jnp.dot(tril, blk, preferred_element_type=jnp.float32)
                         + carry)
            carry = carry + jnp.sum(blk, axis=0, keepdims=True)
        return jnp.concatenate(parts, axis=0), carry

    e1, cnt1 = excl_cumsum(c1)
    e2, cnt2 = excl_cumsum(c2)
    e2 = e2 + cnt1
    counts = cnt1 + cnt2

    pci = (counts.astype(jnp.int32) + (TILE - 1)) & ~jnp.int32(TILE - 1)
    pcf = pci.astype(jnp.float32)
    er = lax.broadcasted_iota(jnp.int32, (E, E), 0)
    ec = lax.broadcasted_iota(jnp.int32, (E, E), 1)
    triu = (er < ec).astype(jnp.float32)
    off = jnp.dot(pcf, triu, preferred_element_type=jnp.float32)

    d0_ref[...] = jnp.sum(c1 * (e1 + off), axis=1, keepdims=True).astype(jnp.int32)
    d1_ref[...] = jnp.sum(c2 * (e2 + off), axis=1, keepdims=True).astype(jnp.int32)
    g0_ref[...] = jnp.broadcast_to(g1, (T, GW))
    g1_ref[...] = jnp.broadcast_to(g2, (T, GW))

    bnd = off + pcf
    jt = (lax.broadcasted_iota(jnp.int32, (1, NT), 1) * TILE).astype(jnp.float32)
    total = bnd[0:1, E - 1:E]
    texp = jnp.zeros((1, NT), jnp.int32)
    elast = jnp.zeros((1, 1), jnp.int32)
    isfirst = jnp.zeros((1, NT), jnp.int32)
    segplus = jnp.zeros((1, NT), jnp.int32)
    for e in range(E):
        b = bnd[0:1, e:e + 1]
        o = off[0:1, e:e + 1]
        pe = pcf[0:1, e:e + 1]
        texp = texp + (b <= jt).astype(jnp.int32)
        elast = elast + (b <= total - 1.0).astype(jnp.int32)
        isfirst = isfirst | ((jt == o) & (pe > 0)).astype(jnp.int32)
        segplus = segplus + ((o <= jt) & (pe > 0)).astype(jnp.int32)
    act = (jt < total).astype(jnp.int32)
    texp = jnp.where(act == 1, jnp.minimum(texp, E - 1), elast)
    isfirst = isfirst * act

    slot = jnp.zeros((1, NT), jnp.int32)
    islot = jnp.zeros((1, NT), jnp.int32)
    for s in range(E):
        slot = slot + jnp.where(segplus == s + 1, s % NSLOT, 0)
        islot = islot + jnp.where(segplus == s + 1, (s + NSLOT - 1) % NSLOT, 0)

    pfe1 = jnp.zeros((1, NT), jnp.int32)
    pfe2 = jnp.zeros((1, NT), jnp.int32)
    nxt_list = []
    for e in range(E):
        nxt = jnp.full((1, 1), E, jnp.int32)
        for e2 in range(E - 1, e, -1):
            nxt = jnp.where(pcf[0:1, e2:e2 + 1] > 0, e2, nxt)
        nxt_list.append(nxt)
    for e in range(E):
        nxt2 = jnp.full((1, 1), E, jnp.int32)
        for e2 in range(E):
            nxt2 = jnp.where(nxt_list[e] == e2, nxt_list[e2], nxt2)
        pfe1 = pfe1 + jnp.where(texp == e, nxt_list[e], 0)
        pfe2 = pfe2 + jnp.where(texp == e, nxt2, 0)
    pfv1 = isfirst * (pfe1 < E).astype(jnp.int32)
    pfv2 = isfirst * (pfe2 < E).astype(jnp.int32)
    pfe1 = jnp.minimum(pfe1, E - 1)
    pfe2 = jnp.minimum(pfe2, E - 1)

    meta_ref[0:1, :] = texp
    meta_ref[1:2, :] = act
    meta_ref[2:3, :] = isfirst
    meta_ref[3:4, :] = slot
    meta_ref[4:5, :] = pfe2 if NSLOT >= 3 else pfe1
    meta_ref[5:6, :] = pfv2 if NSLOT >= 3 else pfv1
    meta_ref[6:7, :] = islot
    meta_ref[7:8, :] = pfe1
    meta_ref[8:9, :] = pfv1


def _router(x, wg):
    return pl.pallas_call(
        _router_body,
        out_shape=(
            jax.ShapeDtypeStruct((T, 1), jnp.int32),
            jax.ShapeDtypeStruct((T, 1), jnp.int32),
            jax.ShapeDtypeStruct((T, GW), jnp.float32),
            jax.ShapeDtypeStruct((T, GW), jnp.float32),
            jax.ShapeDtypeStruct((9, NT), jnp.int32),
        ),
    )(x, wg)


def _dispatch_body(x_hbm, g0_hbm, g1_hbm, d0_hbm, d1_hbm, xg_hbm, gp_hbm,
                   idx0, idx1, rows, gr0, gr1, ssem, scs):
    wid = lax.axis_index("s") * 2 + lax.axis_index("c")
    base = wid * TPW
    stage = (
        pltpu.async_copy(d0_hbm.at[pl.ds(base, TPW)], idx0, ssem),
        pltpu.async_copy(d1_hbm.at[pl.ds(base, TPW)], idx1, ssem),
        pltpu.async_copy(x_hbm.at[pl.ds(base, TPW)], rows, ssem),
        pltpu.async_copy(g0_hbm.at[pl.ds(base, TPW)], gr0, ssem),
        pltpu.async_copy(g1_hbm.at[pl.ds(base, TPW)], gr1, ssem),
    )
    for c in stage:
        c.wait()
    scat = (
        pltpu.async_copy(rows, xg_hbm.at[idx0], scs),
        pltpu.async_copy(rows, xg_hbm.at[idx1], scs),
        pltpu.async_copy(gr0, gp_hbm.at[idx0], scs),
        pltpu.async_copy(gr1, gp_hbm.at[idx1], scs),
    )
    for c in scat:
        c.wait()


def _dispatch(x, g0, g1, d0, d1):
    fn = functools.partial(
        pl.kernel,
        out_type=(
            jax.ShapeDtypeStruct((NR, D_MODEL), jnp.float32),
            jax.ShapeDtypeStruct((NR, GW), jnp.float32),
        ),
        mesh=plsc.VectorSubcoreMesh(core_axis_name="c", subcore_axis_name="s"),
        scratch_types=[
            pltpu.VMEM((TPW,), jnp.int32),
            pltpu.VMEM((TPW,), jnp.int32),
            pltpu.VMEM((TPW, D_MODEL), jnp.float32),
            pltpu.VMEM((TPW, GW), jnp.float32),
            pltpu.VMEM((TPW, GW), jnp.float32),
            pltpu.SemaphoreType.DMA,
            pltpu.SemaphoreType.DMA,
        ],
    )(_dispatch_body)
    return fn(x, g0, g1, d0, d1)


def _ffn_body(te_ref, ta_ref, if_ref, sl_ref, pfe2_ref, pfv2_ref,
              il_ref, pfe1_ref, pfv1_ref,
              xg_ref, w1_hbm, w2_hbm, gp_ref, out_ref,
              w1b, w2b, s1, s2):
    i = pl.program_id(0)

    def start_fetch(expert, slot):
        pltpu.make_async_copy(w1_hbm.at[expert], w1b.at[slot],
                              s1.at[slot]).start()
        pltpu.make_async_copy(w2_hbm.at[expert], w2b.at[slot],
                              s2.at[slot]).start()

    def wait_fetch(slot):
        pltpu.make_async_copy(w1_hbm.at[0], w1b.at[slot], s1.at[slot]).wait()
        pltpu.make_async_copy(w2_hbm.at[0], w2b.at[slot], s2.at[slot]).wait()

    @pl.when(i == 0)
    def _():
        start_fetch(te_ref[0], 0)

    if NSLOT >= 3:
        @pl.when((i == 0) & (pfv1_ref[0] == 1))
        def _():
            start_fetch(pfe1_ref[0], 1)

    for sl in range(NSLOT):
        @pl.when((pfv2_ref[i] == 1) & (il_ref[i] == sl))
        def _(sl=sl):
            start_fetch(pfe2_ref[i], sl)

    for sl in range(NSLOT):
        @pl.when((if_ref[i] == 1) & (sl_ref[i] == sl))
        def _(sl=sl):
            wait_fetch(sl)

    def compute(slot):
        h = jnp.maximum(
            jnp.dot(xg_ref[...], w1b[slot],
                    preferred_element_type=jnp.float32), 0.0)
        out = jnp.dot(h, w2b[slot], preferred_element_type=jnp.float32)
        out_ref[...] = out * gp_ref[:, 0:1]

    for sl in range(NSLOT):
        @pl.when((ta_ref[i] == 1) & (sl_ref[i] == sl))
        def _(sl=sl):
            compute(sl)


def _ffn(meta, xg, w1, w2, gp):
    grid_spec = pltpu.PrefetchScalarGridSpec(
        num_scalar_prefetch=9,
        grid=(NT,),
        in_specs=[
            pl.BlockSpec((TILE, D_MODEL), lambda i, *_: (i, 0)),
            pl.BlockSpec(memory_space=pltpu.MemorySpace.HBM),
            pl.BlockSpec(memory_space=pltpu.MemorySpace.HBM),
            pl.BlockSpec((TILE, GW), lambda i, *_: (i, 0)),
        ],
        out_specs=pl.BlockSpec((TILE, D_MODEL), lambda i, *_: (i, 0)),
        scratch_shapes=[
            pltpu.VMEM((NSLOT, D_MODEL, D_FF), jnp.float32),
            pltpu.VMEM((NSLOT, D_FF, D_MODEL), jnp.float32),
            pltpu.SemaphoreType.DMA((NSLOT,)),
            pltpu.SemaphoreType.DMA((NSLOT,)),
        ],
    )
    return pl.pallas_call(
        _ffn_body,
        grid_spec=grid_spec,
        out_shape=jax.ShapeDtypeStruct((NR, D_MODEL), jnp.float32),
    )(*[meta[r] for r in range(9)], xg, w1, w2, gp)


SUB = 64


SUBC = 16


def _combine_body(out_hbm, d0_hbm, d1_hbm, y_hbm,
                  i0a, i0b, i1a, i1b, r0a, r0b, r1a, r1b,
                  isa, isb, gsa, gsb, osa, osb):
    wid = lax.axis_index("s") * 2 + lax.axis_index("c")
    idx0 = (i0a, i0b)
    idx1 = (i1a, i1b)
    r0 = (r0a, r0b)
    r1 = (r1a, r1b)
    isem = (isa, isb)
    gsem = (gsa, gsb)
    osem = (osa, osb)
    nch = TPW // SUBC

    def stage_idx(k):
        base = wid * TPW + k * SUBC
        p = k % 2
        return (pltpu.async_copy(d0_hbm.at[pl.ds(base, SUBC)], idx0[p], isem[p]),
                pltpu.async_copy(d1_hbm.at[pl.ds(base, SUBC)], idx1[p], isem[p]))

    def start_gather(k, staged):
        p = k % 2
        for c in staged:
            c.wait()
        return (pltpu.async_copy(out_hbm.at[idx0[p]], r0[p], gsem[p]),
                pltpu.async_copy(out_hbm.at[idx1[p]], r1[p], gsem[p]))

    st = stage_idx(0)
    g = start_gather(0, st)
    st_next = stage_idx(1)
    gathers = {0: g}
    for k in range(nch):
        p = k % 2
        if k + 1 < nch:
            if k >= 1:
                pltpu.make_async_copy(r0[1 - p], y_hbm.at[pl.ds(0, SUBC)],
                                      osem[1 - p]).wait()
            gathers[k + 1] = start_gather(k + 1, st_next)
        for c in gathers[k]:
            c.wait()
        if k + 2 < nch:
            st_next = stage_idx(k + 2)

        def col(v, _):
            s = pl.ds(v * 16, 16)
            for j in range(SUBC):
                r0[p][j, s] = r0[p][j, s] + r1[p][j, s]
            return 0

        lax.fori_loop(0, D_MODEL // 16, col, 0, unroll=2)
        base = wid * TPW + k * SUBC
        pltpu.async_copy(r0[p], y_hbm.at[pl.ds(base, SUBC)], osem[p])
    for p in ((nch - 2) % 2, (nch - 1) % 2):
        pltpu.make_async_copy(r0[p], y_hbm.at[pl.ds(0, SUBC)], osem[p]).wait()


def _combine(out, d0, d1):
    fn = functools.partial(
        pl.kernel,
        out_type=jax.ShapeDtypeStruct((T, D_MODEL), jnp.float32),
        mesh=plsc.VectorSubcoreMesh(core_axis_name="c", subcore_axis_name="s"),
        scratch_types=[
            pltpu.VMEM((SUBC,), jnp.int32),
            pltpu.VMEM((SUBC,), jnp.int32),
            pltpu.VMEM((SUBC,), jnp.int32),
            pltpu.VMEM((SUBC,), jnp.int32),
            pltpu.VMEM((SUBC, D_MODEL), jnp.float32),
            pltpu.VMEM((SUBC, D_MODEL), jnp.float32),
            pltpu.VMEM((SUBC, D_MODEL), jnp.float32),
            pltpu.VMEM((SUBC, D_MODEL), jnp.float32),
            pltpu.SemaphoreType.DMA,
            pltpu.SemaphoreType.DMA,
            pltpu.SemaphoreType.DMA,
            pltpu.SemaphoreType.DMA,
            pltpu.SemaphoreType.DMA,
            pltpu.SemaphoreType.DMA,
        ],
    )(_combine_body)
    return fn(out, d0, d1)


def kernel(x, Wg, W1, W2):
    d0, d1, g0, g1, meta = _router(x, Wg)
    d0 = d0.reshape(T)
    d1 = d1.reshape(T)
    xg, gp = _dispatch(x, g0, g1, d0, d1)
    out = _ffn(meta, xg, W1, W2, gp)
    return _combine(out, d0, d1)
